# Initial kernel scaffold; baseline (speedup 1.0000x reference)
import jax
import jax.numpy as jnp
from jax.experimental import pallas as pl


def kernel(images, pixel_data_where, pixel_edge_index, pixel_batch, graph_edge_index, graph_batch, params):
    raise NotImplementedError("write your pallas kernel here")


# speedup vs baseline: 1.1496x; 1.1496x over previous
import functools

import jax
import jax.numpy as jnp
from jax.experimental import pallas as pl
from jax.experimental.pallas import tpu as pltpu

N_PIXEL = 65536
G1 = 2048
NGRAPH = 16
EPS = 1e-5


def _mm_kernel(x_ref, w_ref, b_ref, o_ref):
    o_ref[...] = jnp.dot(x_ref[...], w_ref[...],
                         preferred_element_type=jnp.float32) + b_ref[...]


def _pallas_mm(x, W, b, block_rows=2048):
    n, k = x.shape
    o = W.shape[0]
    wt = W.T
    grid = (n // block_rows,) if n % block_rows == 0 else None
    if grid is None:
        return x @ wt + b
    return pl.pallas_call(
        _mm_kernel,
        grid=grid,
        in_specs=[
            pl.BlockSpec((block_rows, k), lambda i: (i, 0)),
            pl.BlockSpec((k, o), lambda i: (0, 0)),
            pl.BlockSpec((o,), lambda i: (0,)),
        ],
        out_specs=pl.BlockSpec((block_rows, o), lambda i: (i, 0)),
        out_shape=jax.ShapeDtypeStruct((n, o), jnp.float32),
    )(x, wt, b)


def _conv_block(x, p):
    W, b, g, be = p
    out = jax.lax.conv_general_dilated(
        x, W, window_strides=(1, 1), padding=((1, 1), (1, 1)),
        dimension_numbers=('NCHW', 'OIHW', 'NCHW'))
    out = out + b[None, :, None, None]
    m = out.mean(axis=(0, 2, 3))
    v = out.var(axis=(0, 2, 3))
    out = (out - m[None, :, None, None]) / jnp.sqrt(v + EPS)[None, :, None, None] \
        * g[None, :, None, None] + be[None, :, None, None]
    return jax.nn.relu(out)


def _gcn_conv(x, W, b, row, col, n, dis):
    xw = _pallas_mm(x, W, jnp.zeros((W.shape[0],), jnp.float32))
    norm = dis[row] * dis[col]
    out = jnp.zeros((n, xw.shape[1]), xw.dtype).at[col].add(xw[row] * norm[:, None])
    out = out + dis[:, None] * dis[:, None] * xw
    return out + b


def _bn1d(x, g, be):
    m = x.mean(axis=0)
    v = x.var(axis=0)
    return (x - m) / jnp.sqrt(v + EPS) * g + be


def _gmp(x, seg, n):
    s = jax.ops.segment_sum(x, seg, num_segments=n)
    cnt = jax.ops.segment_sum(jnp.ones((x.shape[0],), x.dtype), seg, num_segments=n)
    return s / jnp.clip(cnt, 1.0)[:, None]


def _degrees(row, col, n):
    deg = jnp.zeros((n,), jnp.float32).at[col].add(1.0) + 1.0
    return jnp.where(deg > 0, 1.0 / jnp.sqrt(deg), 0.0)


def _gnn_stack(x, emb, gcns, bns, row, col, n):
    dis = _degrees(row, col, n)
    h = _pallas_mm(x, emb[0], emb[1])
    for (W, b), (g, be) in zip(gcns, bns):
        h = _gcn_conv(h, W, b, row, col, n, dis)
        h = _bn1d(h, g, be)
        h = jax.nn.relu(h)
    return h


def kernel(images, pixel_data_where, pixel_edge_index, pixel_batch,
           graph_edge_index, graph_batch, params):
    x = images
    for p in params['conv']:
        x = _conv_block(x, p)
    feats = x[pixel_data_where[:, 0], :, pixel_data_where[:, 1], pixel_data_where[:, 2]]
    h = _gnn_stack(feats, params['emb1'], params['gcn1'], params['bn1'],
                   pixel_edge_index[0], pixel_edge_index[1], N_PIXEL)
    hg1 = _gmp(h, pixel_batch, G1)
    h = _gnn_stack(hg1, params['emb2'], params['gcn2'], params['bn2'],
                   graph_edge_index[0], graph_edge_index[1], G1)
    hg = _gmp(h, graph_batch, NGRAPH)
    y = hg
    nl = len(params['mlp'])
    for i, (W, b) in enumerate(params['mlp']):
        y = y @ W.T + b
        if i < nl - 1:
            y = jax.nn.relu(y)
    return y
